# baseline (speedup 1.0000x reference)
import functools

import jax
import jax.numpy as jnp
import numpy as np
from jax import lax
from jax.experimental import pallas as pl
from jax.experimental.pallas import tpu as pltpu

_A = 9
_STRIDE = 16
_PRE = 6000
_POST = 300
_THRESH = 0.7
_NEG = -1e30


def _host_anchors(feature_h, feature_w):
    base_size = 16.0
    ratios = np.array([0.5, 1.0, 2.0])
    scales = np.array([8.0, 16.0, 32.0])
    x_ctr = (base_size - 1.0) / 2.0
    y_ctr = (base_size - 1.0) / 2.0
    size = base_size * base_size
    rows = []
    for r in ratios:
        ws = np.round(np.sqrt(size / r))
        hs = np.round(ws * r)
        for s in scales:
            w = ws * s
            h = hs * s
            rows.append([x_ctr - 0.5 * (w - 1.0), y_ctr - 0.5 * (h - 1.0),
                         x_ctr + 0.5 * (w - 1.0), y_ctr + 0.5 * (h - 1.0)])
    base = np.asarray(rows, dtype=np.float32)
    shift_x = np.arange(feature_w, dtype=np.float32) * _STRIDE
    shift_y = np.arange(feature_h, dtype=np.float32) * _STRIDE
    sx, sy = np.meshgrid(shift_x, shift_y)
    shifts = np.stack([sx.ravel(), sy.ravel(), sx.ravel(), sy.ravel()], axis=1).astype(np.float32)
    return (shifts[:, None, :] + base[None, :, :]).reshape(-1, 4)


def _body(sc_ref, dx_ref, dy_ref, dw_ref, dh_ref,
          ax1_ref, ay1_ref, ax2_ref, ay2_ref, h_ref, w_ref,
          out_ref,
          x1_s, y1_s, x2_s, y2_s, ar_s, msc_s, key_s):
    B8, H2 = sc_ref.shape
    B = B8 // 2
    N = 2 * H2
    subl = lax.broadcasted_iota(jnp.int32, (B8, 1), 0)
    iota = (lax.broadcasted_iota(jnp.int32, (B8, H2), 1)
            + jnp.where(subl >= B, jnp.int32(H2), 0))

    def comb_min(v8):
        return jnp.minimum(lax.slice(v8, (0, 0), (B, 1)),
                           lax.slice(v8, (B, 0), (B8, 1)))

    def comb_max(v8):
        return jnp.maximum(lax.slice(v8, (0, 0), (B, 1)),
                           lax.slice(v8, (B, 0), (B8, 1)))

    def comb_sum(v8):
        return (lax.slice(v8, (0, 0), (B, 1))
                + lax.slice(v8, (B, 0), (B8, 1)))

    def up(v4):
        return jnp.concatenate([v4, v4], axis=0)

    ax1 = ax1_ref[...]
    ay1 = ay1_ref[...]
    ax2 = ax2_ref[...]
    ay2 = ay2_ref[...]
    aw = ax2 - ax1 + 1.0
    ah = ay2 - ay1 + 1.0
    acx = ax1 + 0.5 * aw
    acy = ay1 + 0.5 * ah
    pcx = dx_ref[...] * aw + acx
    pcy = dy_ref[...] * ah + acy
    pw = jnp.exp(dw_ref[...]) * aw
    ph = jnp.exp(dh_ref[...]) * ah
    px1 = pcx - 0.5 * pw
    py1 = pcy - 0.5 * ph
    px2 = pcx + 0.5 * pw
    py2 = pcy + 0.5 * ph
    hh = h_ref[...]
    ww = w_ref[...]
    x1 = jnp.clip(px1, 0.0, ww - 1.0)
    y1 = jnp.clip(py1, 0.0, hh - 1.0)
    x2 = jnp.clip(px2, 0.0, ww - 1.0)
    y2 = jnp.clip(py2, 0.0, hh - 1.0)
    x1_s[...] = x1
    y1_s[...] = y1
    x2_s[...] = x2
    y2_s[...] = y2
    ar_s[...] = (x2 - x1 + 1.0) * (y2 - y1 + 1.0)

    sc = sc_ref[...]
    bits = lax.bitcast_convert_type(sc, jnp.int32)
    key = jnp.where(bits < 0, bits ^ jnp.int32(0x7FFFFFFF), bits)
    key_s[...] = key
    min32 = jnp.int32(-2147483648)

    def bs_val(i, u):
        b = 31 - i
        cand_u = u | jnp.left_shift(jnp.int32(1), b)
        thr8 = up(min32 + cand_u)
        cnt = comb_sum(jnp.sum((key_s[...] >= thr8).astype(jnp.int32),
                               axis=1, keepdims=True))
        return jnp.where(cnt >= _PRE, cand_u, u)

    u = lax.fori_loop(0, 32, bs_val, jnp.zeros((B, 1), jnp.int32))
    t6 = min32 + u
    t68 = up(t6)

    keyv = key_s[...]
    cnt_gt = comb_sum(jnp.sum((keyv > t68).astype(jnp.int32),
                              axis=1, keepdims=True))
    quota = _PRE - cnt_gt

    def bs_idx(i, lohi):
        lo, hi = lohi
        mid = (lo + hi) >> 1
        mid8 = up(mid)
        g = comb_sum(jnp.sum(((key_s[...] == t68) & (iota <= mid8))
                             .astype(jnp.int32), axis=1, keepdims=True))
        ok = g >= quota
        return jnp.where(ok, lo, mid + 1), jnp.where(ok, mid, hi)

    lo0 = jnp.zeros((B, 1), jnp.int32)
    hi0 = jnp.full((B, 1), N - 1, jnp.int32)
    _, bound = lax.fori_loop(0, 17, bs_idx, (lo0, hi0))

    sel = (keyv > t68) | ((keyv == t68) & (iota <= up(bound)))
    msc_s[...] = jnp.where(sel, sc, jnp.float32(_NEG))

    lane = lax.broadcasted_iota(jnp.int32, (1, 128), 1)
    fill = jnp.float32(-3.0e38)

    def chosen(idx):
        rows = [[], [], [], []]
        for b in range(B):
            ib = jnp.sum(lax.slice(idx, (b, 0), (b + 1, 1)))
            half = ib // H2
            ibl = ib - half * H2
            cb = pl.multiple_of((ibl // 128) * 128, 128)
            lb = ibl - cb
            selc = lane == lb
            hf = half.astype(jnp.float32)
            for t, ref in enumerate((x1_s, y1_s, x2_s, y2_s)):
                ch0 = ref[pl.ds(b, 1), pl.ds(cb, 128)]
                ch1 = ref[pl.ds(b + B, 1), pl.ds(cb, 128)]
                ch = ch0 * (1.0 - hf) + ch1 * hf
                rows[t].append(jnp.max(jnp.where(selc, ch, fill),
                                       axis=1, keepdims=True))
        return [jnp.concatenate(r, axis=0) for r in rows]

    def argmax(msc):
        m = comb_max(jnp.max(msc, axis=1, keepdims=True))
        idx = comb_min(jnp.min(jnp.where(msc == up(m), iota, jnp.int32(N)),
                               axis=1, keepdims=True))
        return m, idx

    def nms_cond(carry):
        emitted = carry
        return jnp.sum(jnp.min(emitted)) < _POST

    def nms(carry):
        emitted = carry
        msc = msc_s[...]
        m1, idx1 = argmax(msc)
        selm1 = iota == up(idx1)
        msc_no1 = jnp.where(selm1, jnp.float32(_NEG), msc)
        m2, idx2 = argmax(msc_no1)
        selm2 = iota == up(idx2)

        cx1, cy1, cx2, cy2 = chosen(idx1)
        dx1, dy1, dx2, dy2 = chosen(idx2)
        carea1 = (cx2 - cx1 + 1.0) * (cy2 - cy1 + 1.0)
        carea2 = (dx2 - dx1 + 1.0) * (dy2 - dy1 + 1.0)

        pxx1 = jnp.maximum(cx1, dx1)
        pyy1 = jnp.maximum(cy1, dy1)
        pxx2 = jnp.minimum(cx2, dx2)
        pyy2 = jnp.minimum(cy2, dy2)
        piw = jnp.maximum(pxx2 - pxx1 + 1.0, 0.0)
        pih = jnp.maximum(pyy2 - pyy1 + 1.0, 0.0)
        pint = piw * pih
        piou = pint / (carea1 + carea2 - pint)
        dbl_i = jnp.where(piou > _THRESH, jnp.int32(0), jnp.int32(1))
        dbl8 = up(dbl_i) > 0

        xx1 = jnp.maximum(up(cx1), x1_s[...])
        yy1 = jnp.maximum(up(cy1), y1_s[...])
        xx2 = jnp.minimum(up(cx2), x2_s[...])
        yy2 = jnp.minimum(up(cy2), y2_s[...])
        iw = jnp.maximum(xx2 - xx1 + 1.0, 0.0)
        ih = jnp.maximum(yy2 - yy1 + 1.0, 0.0)
        inter = iw * ih
        iou1 = inter / (up(carea1) + ar_s[...] - inter)

        qx1 = jnp.maximum(up(dx1), x1_s[...])
        qy1 = jnp.maximum(up(dy1), y1_s[...])
        qx2 = jnp.minimum(up(dx2), x2_s[...])
        qy2 = jnp.minimum(up(dy2), y2_s[...])
        qw = jnp.maximum(qx2 - qx1 + 1.0, 0.0)
        qh = jnp.maximum(qy2 - qy1 + 1.0, 0.0)
        qint = qw * qh
        iou2 = qint / (up(carea2) + ar_s[...] - qint)

        supp = (iou1 > _THRESH) | selm1 | (((iou2 > _THRESH) | selm2) & dbl8)
        msc_s[...] = jnp.where(supp, jnp.float32(_NEG), msc)

        valid1 = (m1 > jnp.float32(_NEG * 0.5)).astype(jnp.float32)
        valid2 = (m2 > jnp.float32(_NEG * 0.5)).astype(jnp.float32)
        for b in range(B):
            pos = jnp.sum(lax.slice(emitted, (b, 0), (b + 1, 1)))
            db = jnp.sum(lax.slice(dbl_i, (b, 0), (b + 1, 1)))
            bf = jnp.float32(b)

            def rowvec(v, cols):
                vb = jnp.sum(lax.slice(v, (b, 0), (b + 1, 1)))
                parts = [jnp.full((1, 1), bf, jnp.float32)]
                for cc in cols:
                    cb_ = jnp.sum(lax.slice(cc, (b, 0), (b + 1, 1)))
                    parts.append(
                        jax.lax.broadcast_in_dim(cb_ * vb, (1, 1), ()))
                return jnp.concatenate(parts, axis=1)

            @pl.when(pos < _POST)
            def _():
                out_ref[pos, pl.ds(b, 1), :] = rowvec(
                    valid1, (cx1, cy1, cx2, cy2))

            @pl.when((db == 1) & (pos + 1 < _POST))
            def _():
                out_ref[pos + 1, pl.ds(b, 1), :] = rowvec(
                    valid2, (dx1, dy1, dx2, dy2))

        return jnp.minimum(emitted + 1 + dbl_i, _POST)

    lax.while_loop(nms_cond, nms, jnp.zeros((B, 1), jnp.int32))


@functools.partial(jax.jit, static_argnames=())
def kernel(scores_raw, bbox_deltas, im_info):
    B = scores_raw.shape[0]
    H, W = scores_raw.shape[2], scores_raw.shape[3]
    N = H * W * _A
    H2 = N // 2
    f32 = jnp.float32

    def split(a):
        return jnp.concatenate([a[:, :H2], a[:, H2:]], axis=0)

    sc = split(jnp.transpose(scores_raw[:, _A:], (0, 2, 3, 1)).reshape(B, N))
    d = jnp.transpose(bbox_deltas, (0, 2, 3, 1)).reshape(B, N, 4)
    dx, dy, dw, dh = (split(d[..., 0]), split(d[..., 1]),
                      split(d[..., 2]), split(d[..., 3]))

    anch = _host_anchors(H, W)

    def asplit(col):
        a2 = jnp.asarray(col).reshape(2, H2)
        return jnp.repeat(a2, B, axis=0)

    ax1 = asplit(anch[:, 0])
    ay1 = asplit(anch[:, 1])
    ax2 = asplit(anch[:, 2])
    ay2 = asplit(anch[:, 3])
    hcol = jnp.tile(im_info[:, 0:1].astype(f32), (2, 1))
    wcol = jnp.tile(im_info[:, 1:2].astype(f32), (2, 1))

    out = pl.pallas_call(
        _body,
        out_shape=jax.ShapeDtypeStruct((_POST, B, 5), f32),
        in_specs=[pl.BlockSpec(memory_space=pltpu.VMEM)] * 11,
        out_specs=pl.BlockSpec(memory_space=pltpu.VMEM),
        scratch_shapes=[
            pltpu.VMEM((2 * B, H2), f32),
            pltpu.VMEM((2 * B, H2), f32),
            pltpu.VMEM((2 * B, H2), f32),
            pltpu.VMEM((2 * B, H2), f32),
            pltpu.VMEM((2 * B, H2), f32),
            pltpu.VMEM((2 * B, H2), f32),
            pltpu.VMEM((2 * B, H2), jnp.int32),
        ],
    )(sc, dx, dy, dw, dh, ax1, ay1, ax2, ay2, hcol, wcol)
    return jnp.transpose(out, (1, 0, 2))


# speedup vs baseline: 1.1107x; 1.1107x over previous
import functools

import jax
import jax.numpy as jnp
import numpy as np
from jax import lax
from jax.experimental import pallas as pl
from jax.experimental.pallas import tpu as pltpu

_A = 9
_STRIDE = 16
_PRE = 6000
_POST = 300
_THRESH = 0.7
_NEG = -1e30


def _host_anchors(feature_h, feature_w):
    base_size = 16.0
    ratios = np.array([0.5, 1.0, 2.0])
    scales = np.array([8.0, 16.0, 32.0])
    x_ctr = (base_size - 1.0) / 2.0
    y_ctr = (base_size - 1.0) / 2.0
    size = base_size * base_size
    rows = []
    for r in ratios:
        ws = np.round(np.sqrt(size / r))
        hs = np.round(ws * r)
        for s in scales:
            w = ws * s
            h = hs * s
            rows.append([x_ctr - 0.5 * (w - 1.0), y_ctr - 0.5 * (h - 1.0),
                         x_ctr + 0.5 * (w - 1.0), y_ctr + 0.5 * (h - 1.0)])
    base = np.asarray(rows, dtype=np.float32)
    shift_x = np.arange(feature_w, dtype=np.float32) * _STRIDE
    shift_y = np.arange(feature_h, dtype=np.float32) * _STRIDE
    sx, sy = np.meshgrid(shift_x, shift_y)
    shifts = np.stack([sx.ravel(), sy.ravel(), sx.ravel(), sy.ravel()], axis=1).astype(np.float32)
    return (shifts[:, None, :] + base[None, :, :]).reshape(-1, 4)


def _body(sc_ref, dx_ref, dy_ref, dw_ref, dh_ref,
          ax1_ref, ay1_ref, ax2_ref, ay2_ref, h_ref, w_ref,
          out_ref,
          x1_s, y1_s, x2_s, y2_s, ar_s, msc_s, key_s):
    B8, H2 = sc_ref.shape
    B = B8 // 2
    N = 2 * H2
    subl = lax.broadcasted_iota(jnp.int32, (B8, 1), 0)
    iota = (lax.broadcasted_iota(jnp.int32, (B8, H2), 1)
            + jnp.where(subl >= B, jnp.int32(H2), 0))

    def comb_min(v8):
        return jnp.minimum(lax.slice(v8, (0, 0), (B, 1)),
                           lax.slice(v8, (B, 0), (B8, 1)))

    def comb_max(v8):
        return jnp.maximum(lax.slice(v8, (0, 0), (B, 1)),
                           lax.slice(v8, (B, 0), (B8, 1)))

    def comb_sum(v8):
        return (lax.slice(v8, (0, 0), (B, 1))
                + lax.slice(v8, (B, 0), (B8, 1)))

    def up(v4):
        return jnp.concatenate([v4, v4], axis=0)

    ax1 = ax1_ref[...]
    ay1 = ay1_ref[...]
    ax2 = ax2_ref[...]
    ay2 = ay2_ref[...]
    aw = ax2 - ax1 + 1.0
    ah = ay2 - ay1 + 1.0
    acx = ax1 + 0.5 * aw
    acy = ay1 + 0.5 * ah
    pcx = dx_ref[...] * aw + acx
    pcy = dy_ref[...] * ah + acy
    pw = jnp.exp(dw_ref[...]) * aw
    ph = jnp.exp(dh_ref[...]) * ah
    px1 = pcx - 0.5 * pw
    py1 = pcy - 0.5 * ph
    px2 = pcx + 0.5 * pw
    py2 = pcy + 0.5 * ph
    hh = h_ref[...]
    ww = w_ref[...]
    x1 = jnp.clip(px1, 0.0, ww - 1.0)
    y1 = jnp.clip(py1, 0.0, hh - 1.0)
    x2 = jnp.clip(px2, 0.0, ww - 1.0)
    y2 = jnp.clip(py2, 0.0, hh - 1.0)
    x1_s[...] = x1
    y1_s[...] = y1
    x2_s[...] = x2
    y2_s[...] = y2
    ar_s[...] = (x2 - x1 + 1.0) * (y2 - y1 + 1.0)

    sc = sc_ref[...]
    bits = lax.bitcast_convert_type(sc, jnp.int32)
    key = jnp.where(bits < 0, bits ^ jnp.int32(0x7FFFFFFF), bits)
    key_s[...] = key
    min32 = jnp.int32(-2147483648)

    def bs_val(i, u):
        b = 31 - i
        cand_u = u | jnp.left_shift(jnp.int32(1), b)
        thr8 = up(min32 + cand_u)
        cnt = comb_sum(jnp.sum((key_s[...] >= thr8).astype(jnp.int32),
                               axis=1, keepdims=True))
        return jnp.where(cnt >= _PRE, cand_u, u)

    u = lax.fori_loop(0, 32, bs_val, jnp.zeros((B, 1), jnp.int32))
    t6 = min32 + u
    t68 = up(t6)

    keyv = key_s[...]
    cnt_gt = comb_sum(jnp.sum((keyv > t68).astype(jnp.int32),
                              axis=1, keepdims=True))
    quota = _PRE - cnt_gt

    def bs_idx(i, lohi):
        lo, hi = lohi
        mid = (lo + hi) >> 1
        mid8 = up(mid)
        g = comb_sum(jnp.sum(((key_s[...] == t68) & (iota <= mid8))
                             .astype(jnp.int32), axis=1, keepdims=True))
        ok = g >= quota
        return jnp.where(ok, lo, mid + 1), jnp.where(ok, mid, hi)

    lo0 = jnp.zeros((B, 1), jnp.int32)
    hi0 = jnp.full((B, 1), N - 1, jnp.int32)
    _, bound = lax.fori_loop(0, 17, bs_idx, (lo0, hi0))

    sel = (keyv > t68) | ((keyv == t68) & (iota <= up(bound)))
    msc_s[...] = jnp.where(sel, sc, jnp.float32(_NEG))

    bcol = lax.broadcasted_iota(jnp.int32, (B, 1), 0).astype(jnp.float32)
    lane = lax.broadcasted_iota(jnp.int32, (1, 128), 1)

    def nms(i, _):
        msc = msc_s[...]
        m = comb_max(jnp.max(msc, axis=1, keepdims=True))
        m8 = up(m)
        idx = comb_min(jnp.min(jnp.where(msc == m8, iota, jnp.int32(N)),
                               axis=1, keepdims=True))
        idx8 = up(idx)
        selm = iota == idx8
        fill = jnp.float32(-3.0e38)
        rows = [[], [], [], []]
        for b in range(B):
            ib = jnp.sum(lax.slice(idx, (b, 0), (b + 1, 1)))
            half = ib // H2
            ibl = ib - half * H2
            cb = pl.multiple_of((ibl // 128) * 128, 128)
            lb = ibl - cb
            selc = lane == lb
            hf = half.astype(jnp.float32)
            for t, ref in enumerate((x1_s, y1_s, x2_s, y2_s)):
                ch0 = ref[pl.ds(b, 1), pl.ds(cb, 128)]
                ch1 = ref[pl.ds(b + B, 1), pl.ds(cb, 128)]
                ch = ch0 * (1.0 - hf) + ch1 * hf
                rows[t].append(jnp.max(jnp.where(selc, ch, fill),
                                       axis=1, keepdims=True))
        cx1 = jnp.concatenate(rows[0], axis=0)
        cy1 = jnp.concatenate(rows[1], axis=0)
        cx2 = jnp.concatenate(rows[2], axis=0)
        cy2 = jnp.concatenate(rows[3], axis=0)
        carea = (cx2 - cx1 + 1.0) * (cy2 - cy1 + 1.0)
        valid = (m > jnp.float32(_NEG * 0.5)).astype(jnp.float32)
        xx1 = jnp.maximum(up(cx1), x1_s[...])
        yy1 = jnp.maximum(up(cy1), y1_s[...])
        xx2 = jnp.minimum(up(cx2), x2_s[...])
        yy2 = jnp.minimum(up(cy2), y2_s[...])
        iw = jnp.maximum(xx2 - xx1 + 1.0, 0.0)
        ih = jnp.maximum(yy2 - yy1 + 1.0, 0.0)
        inter = iw * ih
        iou = inter / (up(carea) + ar_s[...] - inter)
        msc_s[...] = jnp.where((iou > _THRESH) | selm, jnp.float32(_NEG), msc)
        row = jnp.concatenate(
            [bcol, cx1 * valid, cy1 * valid, cx2 * valid, cy2 * valid], axis=1)
        out_ref[i, :, :] = row
        return 0

    lax.fori_loop(0, _POST, nms, 0)


@functools.partial(jax.jit, static_argnames=())
def kernel(scores_raw, bbox_deltas, im_info):
    B = scores_raw.shape[0]
    H, W = scores_raw.shape[2], scores_raw.shape[3]
    N = H * W * _A
    H2 = N // 2
    f32 = jnp.float32

    def split(a):
        return jnp.concatenate([a[:, :H2], a[:, H2:]], axis=0)

    sc = split(jnp.transpose(scores_raw[:, _A:], (0, 2, 3, 1)).reshape(B, N))
    d = jnp.transpose(bbox_deltas, (0, 2, 3, 1)).reshape(B, N, 4)
    dx, dy, dw, dh = (split(d[..., 0]), split(d[..., 1]),
                      split(d[..., 2]), split(d[..., 3]))

    anch = _host_anchors(H, W)

    def asplit(col):
        a2 = jnp.asarray(col).reshape(2, H2)
        return jnp.repeat(a2, B, axis=0)

    ax1 = asplit(anch[:, 0])
    ay1 = asplit(anch[:, 1])
    ax2 = asplit(anch[:, 2])
    ay2 = asplit(anch[:, 3])
    hcol = jnp.tile(im_info[:, 0:1].astype(f32), (2, 1))
    wcol = jnp.tile(im_info[:, 1:2].astype(f32), (2, 1))

    out = pl.pallas_call(
        _body,
        out_shape=jax.ShapeDtypeStruct((_POST, B, 5), f32),
        in_specs=[pl.BlockSpec(memory_space=pltpu.VMEM)] * 11,
        out_specs=pl.BlockSpec(memory_space=pltpu.VMEM),
        scratch_shapes=[
            pltpu.VMEM((2 * B, H2), f32),
            pltpu.VMEM((2 * B, H2), f32),
            pltpu.VMEM((2 * B, H2), f32),
            pltpu.VMEM((2 * B, H2), f32),
            pltpu.VMEM((2 * B, H2), f32),
            pltpu.VMEM((2 * B, H2), f32),
            pltpu.VMEM((2 * B, H2), jnp.int32),
        ],
    )(sc, dx, dy, dw, dh, ax1, ay1, ax2, ay2, hcol, wcol)
    return jnp.transpose(out, (1, 0, 2))
